# Initial kernel scaffold; baseline (speedup 1.0000x reference)
import jax
import jax.numpy as jnp
from jax.experimental import pallas as pl


def kernel(x, table):
    raise NotImplementedError("write your pallas kernel here")


# speedup vs baseline: 4.1991x; 4.1991x over previous
import functools

import jax
import jax.numpy as jnp
from jax import lax
from jax.experimental import pallas as pl
from jax.experimental.pallas import tpu as pltpu

_CORNER_RES = 64
_PI2 = 19
_PI3 = 389


def _gather_kernel(x_ref, tbl_hbm, out_ref, tbl_vmem, sem, *, tmask, npts):
    cp = pltpu.make_async_copy(tbl_hbm, tbl_vmem, sem)
    cp.start()
    cp.wait()

    key_iota = (
        (lax.broadcasted_iota(jnp.int32, (8, 128), 0) << 2)
        | (lax.broadcasted_iota(jnp.int32, (8, 128), 1) >> 5)
    )
    res_f = jnp.float32(_CORNER_RES)

    def chunk_body(c, carry):
        base = c * 8
        rows = []
        for j in range(8):
            i = base + j
            xs0 = x_ref[i, 0] * res_f
            xs1 = x_ref[i, 1] * res_f
            xs2 = x_ref[i, 2] * res_f
            i0 = xs0.astype(jnp.int32)
            i1 = xs1.astype(jnp.int32)
            i2 = xs2.astype(jnp.int32)
            f0 = xs0 - i0.astype(jnp.float32)
            f1 = xs1 - i1.astype(jnp.float32)
            f2 = xs2 - i2.astype(jnp.float32)
            a1 = i1 * _PI2
            a2 = i2 * _PI3
            pxy = (i0 ^ a1, i0 ^ (a1 + _PI2), (i0 + 1) ^ a1, (i0 + 1) ^ (a1 + _PI2))
            wxy = ((1.0 - f0) * (1.0 - f1), (1.0 - f0) * f1,
                   f0 * (1.0 - f1), f0 * f1)
            zs = ((a2, 1.0 - f2), (a2 + _PI3, f2))
            acc = jnp.zeros((8, 128), jnp.float32)
            for q in range(4):
                for az, wz in zs:
                    idx = (pxy[q] ^ az) & tmask
                    w = wxy[q] * wz
                    cb = pl.multiple_of((idx >> 5) << 3, 8)
                    chunk = tbl_vmem[pl.ds(cb, 8), :]
                    acc = acc + jnp.where(key_iota == (idx & 31), w,
                                          jnp.float32(0.0)) * chunk
            rows.append(jnp.sum(acc, axis=0, keepdims=True))
        blk = jnp.concatenate(rows, axis=0)
        r = (blk[:, 0:32] + blk[:, 32:64]) + (blk[:, 64:96] + blk[:, 96:128])
        out_ref[pl.ds(base, 8), :] = r
        return carry

    lax.fori_loop(0, npts // 8, chunk_body, 0)


def kernel(x, table):
    b, d = x.shape
    t, f = table.shape
    assert d == 3 and f == 32
    assert t & (t - 1) == 0 and b % 16 == 0
    packed = table.reshape(t // 4, 128)

    pb = b // 2
    kern = functools.partial(_gather_kernel, tmask=t - 1, npts=pb)

    out = pl.pallas_call(
        kern,
        out_shape=jax.ShapeDtypeStruct((b, f), x.dtype),
        grid=(2,),
        in_specs=[
            pl.BlockSpec((pb, 3), lambda i: (i, 0), memory_space=pltpu.SMEM),
            pl.BlockSpec(memory_space=pl.ANY),
        ],
        out_specs=pl.BlockSpec((pb, f), lambda i: (i, 0)),
        scratch_shapes=[
            pltpu.VMEM((t // 4, 128), jnp.float32),
            pltpu.SemaphoreType.DMA,
        ],
        compiler_params=pltpu.CompilerParams(
            dimension_semantics=("parallel",),
            vmem_limit_bytes=40 << 20,
        ),
    )(x, packed)
    return out


# speedup vs baseline: 7.0391x; 1.6763x over previous
import functools

import jax
import jax.numpy as jnp
from jax import lax
from jax.experimental import pallas as pl
from jax.experimental.pallas import tpu as pltpu

_RES = 64
_PI2 = 19
_PI3 = 389


def _ingp_kernel(x_ref, tbl_hbm, out_ref, slab, wts, sem, *, tmask, npts):
    t = tbl_hbm.shape[0]
    f = tbl_hbm.shape[1]
    tbl3 = tbl_hbm.reshape(t, 1, f)
    res_f = jnp.float32(_RES)

    def issue_body(i, carry):
        xs0 = x_ref[i, 0] * res_f
        xs1 = x_ref[i, 1] * res_f
        xs2 = x_ref[i, 2] * res_f
        i0 = xs0.astype(jnp.int32)
        i1 = xs1.astype(jnp.int32)
        i2 = xs2.astype(jnp.int32)
        f0 = xs0 - i0.astype(jnp.float32)
        f1 = xs1 - i1.astype(jnp.float32)
        f2 = xs2 - i2.astype(jnp.float32)
        a1 = i1 * _PI2
        a2 = i2 * _PI3
        pxy = (i0 ^ a1, i0 ^ (a1 + _PI2), (i0 + 1) ^ a1, (i0 + 1) ^ (a1 + _PI2))
        wxy = ((1.0 - f0) * (1.0 - f1), (1.0 - f0) * f1,
               f0 * (1.0 - f1), f0 * f1)
        zs = ((a2, 1.0 - f2), (a2 + _PI3, f2))
        base = i * 8
        c = 0
        for q in range(4):
            for az, wz in zs:
                idx = (pxy[q] ^ az) & tmask
                wts[i, c] = wxy[q] * wz
                pltpu.make_async_copy(tbl3.at[idx], slab.at[base + c],
                                      sem).start()
                c += 1
        return carry

    lax.fori_loop(0, npts, issue_body, 0)
    pltpu.make_async_copy(slab, slab, sem).wait()

    def chunk_body(cix, carry):
        base = cix * 8
        rows = []
        for j in range(8):
            i = base + j
            acc = wts[i, 0] * slab[i * 8, 0, :]
            for c in range(1, 8):
                acc = acc + wts[i, c] * slab[i * 8 + c, 0, :]
            rows.append(acc.reshape(1, f))
        out_ref[pl.ds(base, 8), :] = jnp.concatenate(rows, axis=0)
        return carry

    lax.fori_loop(0, npts // 8, chunk_body, 0)


def kernel(x, table):
    b, d = x.shape
    t, f = table.shape
    assert d == 3 and f == 32
    assert t & (t - 1) == 0 and b % 16 == 0

    pb = b // 2
    kern = functools.partial(_ingp_kernel, tmask=t - 1, npts=pb)

    out = pl.pallas_call(
        kern,
        out_shape=jax.ShapeDtypeStruct((b, f), x.dtype),
        grid=(2,),
        in_specs=[
            pl.BlockSpec((pb, 3), lambda i: (i, 0), memory_space=pltpu.SMEM),
            pl.BlockSpec(memory_space=pl.ANY),
        ],
        out_specs=pl.BlockSpec((pb, f), lambda i: (i, 0)),
        scratch_shapes=[
            pltpu.VMEM((pb * 8, 1, f), jnp.float32),
            pltpu.SMEM((pb, 8), jnp.float32),
            pltpu.SemaphoreType.DMA,
        ],
        compiler_params=pltpu.CompilerParams(
            dimension_semantics=("parallel",),
            vmem_limit_bytes=32 << 20,
        ),
    )(x, table)
    return out
